# baseline (speedup 1.0000x reference)
import jax
import jax.numpy as jnp
from jax import lax
from jax.experimental import pallas as pl
from jax.experimental.pallas import tpu as pltpu
from jax.experimental.pallas import tpu_sc as plsc

NUM_CORES = 2
NUM_SUBCORES = 16
NUM_WORKERS = NUM_CORES * NUM_SUBCORES
LANES = 16
CHUNK = 32


def _mf_body(uidx_hbm, iidx_hbm, utab_hbm, itab_hbm, out_hbm,
             idx_uv, idx_iv, tbuf_u, tbuf_i, rows_u, rows_i, out_v, prod,
             sem_u, sem_i):
    b_per_w = idx_uv.shape[0]
    embed = utab_hbm.shape[1]
    wid = lax.axis_index("s") * NUM_CORES + lax.axis_index("c")
    base = wid * b_per_w

    pltpu.sync_copy(uidx_hbm.at[pl.ds(base, b_per_w)], idx_uv)
    pltpu.sync_copy(iidx_hbm.at[pl.ds(base, b_per_w)], idx_iv)

    def chunk_body(c, carry):
        c0 = c * CHUNK
        for q in range(CHUNK // LANES):
            vb = c0 + q * LANES
            vu = idx_uv[pl.ds(vb, LANES)]
            vi = idx_iv[pl.ds(vb, LANES)]
            tu_v = vu & jnp.int32(~7)
            ti_v = vi & jnp.int32(~7)
            for j in range(LANES):
                slot = q * LANES + j
                tu = pl.multiple_of((tu_v[j] >> 20) << 3, 8)
                ti = pl.multiple_of((ti_v[j] >> 20) << 3, 8)
                pltpu.async_copy(utab_hbm.at[pl.ds(tu, 8)],
                                 tbuf_u.at[pl.ds(slot * 8, 8)], sem_u)
                pltpu.async_copy(itab_hbm.at[pl.ds(ti, 8)],
                                 tbuf_i.at[pl.ds(slot * 8, 8)], sem_i)
        pltpu.make_async_copy(utab_hbm.at[pl.ds(0, CHUNK * 8)], tbuf_u,
                              sem_u).wait()
        pltpu.make_async_copy(itab_hbm.at[pl.ds(0, CHUNK * 8)], tbuf_i,
                              sem_i).wait()

        for q in range(CHUNK // LANES):
            vb = c0 + q * LANES
            ou_v = idx_uv[pl.ds(vb, LANES)] & 7
            oi_v = idx_iv[pl.ds(vb, LANES)] & 7
            for j in range(LANES):
                slot = q * LANES + j
                b = c0 + slot
                ou = ou_v[j]
                oi = oi_v[j]
                for h in range(embed // LANES):
                    rows_u[pl.ds(b * embed + h * LANES, LANES)] = (
                        tbuf_u[slot * 8 + ou, pl.ds(h * LANES, LANES)])
                    rows_i[pl.ds(b * embed + h * LANES, LANES)] = (
                        tbuf_i[slot * 8 + oi, pl.ds(h * LANES, LANES)])
        return carry

    lax.fori_loop(0, b_per_w // CHUNK, chunk_body, 0)

    def group(g, carry):
        for r in range(LANES):
            b = g * LANES + r
            p = jnp.zeros((LANES,), jnp.float32)
            for h in range(embed // LANES):
                u = rows_u[pl.ds(b * embed + h * LANES, LANES)]
                v = rows_i[pl.ds(b * embed + h * LANES, LANES)]
                p = p + u * v
            prod[pl.ds(r * LANES, LANES)] = p
        stride_idx = LANES * lax.iota(jnp.int32, LANES)
        acc = jnp.zeros((LANES,), jnp.float32)
        for k in range(LANES):
            acc = acc + plsc.load_gather(prod, [stride_idx + k])
        sig = 1.0 / (1.0 + jnp.exp(-acc))
        out_v[pl.ds(g * LANES, LANES)] = sig
        return carry

    lax.fori_loop(0, b_per_w // LANES, group, 0)
    pltpu.sync_copy(out_v, out_hbm.at[pl.ds(base, b_per_w)])


def kernel(x, user_emb_table, item_emb_table):
    batch = x.shape[0]
    embed = user_emb_table.shape[1]
    b_per_w = batch // NUM_WORKERS
    user_idx = x[:, 0]
    item_idx = x[:, 1]
    mesh = plsc.VectorSubcoreMesh(core_axis_name="c", subcore_axis_name="s")
    f = pl.kernel(
        _mf_body,
        out_type=jax.ShapeDtypeStruct((batch,), jnp.float32),
        mesh=mesh,
        compiler_params=pltpu.CompilerParams(needs_layout_passes=False,
                                             use_tc_tiling_on_sc=True),
        scratch_types=[
            pltpu.VMEM((b_per_w,), jnp.int32),
            pltpu.VMEM((b_per_w,), jnp.int32),
            pltpu.VMEM((CHUNK * 8, embed), jnp.float32),
            pltpu.VMEM((CHUNK * 8, embed), jnp.float32),
            pltpu.VMEM((b_per_w * embed,), jnp.float32),
            pltpu.VMEM((b_per_w * embed,), jnp.float32),
            pltpu.VMEM((b_per_w,), jnp.float32),
            pltpu.VMEM((LANES * LANES,), jnp.float32),
            pltpu.SemaphoreType.DMA,
            pltpu.SemaphoreType.DMA,
        ],
    )
    return f(user_idx, item_idx, user_emb_table, item_emb_table)


# speedup vs baseline: 1.8533x; 1.8533x over previous
import jax
import jax.numpy as jnp
from jax import lax
from jax.experimental import pallas as pl
from jax.experimental.pallas import tpu as pltpu
from jax.experimental.pallas import tpu_sc as plsc

NUM_CORES = 2
NUM_SUBCORES = 16
NUM_WORKERS = NUM_CORES * NUM_SUBCORES
LANES = 16
CHUNK = 32


def _mf_body(uidx_hbm, iidx_hbm, utab_hbm, itab_hbm, out_hbm,
             idx_uv, idx_iv, tbuf_u, tbuf_i, rows_u, rows_i, out_v, prod,
             sem_u, sem_i):
    b_per_w = idx_uv.shape[0]
    embed = utab_hbm.shape[1]
    wid = lax.axis_index("s") * NUM_CORES + lax.axis_index("c")
    base = wid * b_per_w

    pltpu.sync_copy(uidx_hbm.at[pl.ds(base, b_per_w)], idx_uv)
    pltpu.sync_copy(iidx_hbm.at[pl.ds(base, b_per_w)], idx_iv)

    def chunk_body(c, carry):
        c0 = c * CHUNK
        for q in range(CHUNK // LANES):
            vb = c0 + q * LANES
            vu = idx_uv[pl.ds(vb, LANES)]
            vi = idx_iv[pl.ds(vb, LANES)]
            tu_v = vu & jnp.int32(~7)
            ti_v = vi & jnp.int32(~7)
            for j in range(LANES):
                slot = q * LANES + j
                tu = pl.multiple_of(tu_v[j], 8)
                ti = pl.multiple_of(ti_v[j], 8)
                pltpu.async_copy(utab_hbm.at[pl.ds(tu, 8)],
                                 tbuf_u.at[pl.ds(slot * 8, 8)], sem_u)
                pltpu.async_copy(itab_hbm.at[pl.ds(ti, 8)],
                                 tbuf_i.at[pl.ds(slot * 8, 8)], sem_i)
        pltpu.make_async_copy(utab_hbm.at[pl.ds(0, CHUNK * 8)], tbuf_u,
                              sem_u).wait()
        pltpu.make_async_copy(itab_hbm.at[pl.ds(0, CHUNK * 8)], tbuf_i,
                              sem_i).wait()

        for q in range(CHUNK // LANES):
            vb = c0 + q * LANES
            ou_v = idx_uv[pl.ds(vb, LANES)] & 7
            oi_v = idx_iv[pl.ds(vb, LANES)] & 7
            for j in range(LANES):
                slot = q * LANES + j
                b = c0 + slot
                ou = ou_v[j]
                oi = oi_v[j]
                for h in range(embed // LANES):
                    rows_u[pl.ds(b * embed + h * LANES, LANES)] = (
                        tbuf_u[slot * 8 + ou, pl.ds(h * LANES, LANES)])
                    rows_i[pl.ds(b * embed + h * LANES, LANES)] = (
                        tbuf_i[slot * 8 + oi, pl.ds(h * LANES, LANES)])
        return carry

    lax.fori_loop(0, b_per_w // CHUNK, chunk_body, 0)

    def group(g, carry):
        for r in range(LANES):
            b = g * LANES + r
            p = jnp.zeros((LANES,), jnp.float32)
            for h in range(embed // LANES):
                u = rows_u[pl.ds(b * embed + h * LANES, LANES)]
                v = rows_i[pl.ds(b * embed + h * LANES, LANES)]
                p = p + u * v
            prod[pl.ds(r * LANES, LANES)] = p
        stride_idx = LANES * lax.iota(jnp.int32, LANES)
        acc = jnp.zeros((LANES,), jnp.float32)
        for k in range(LANES):
            acc = acc + plsc.load_gather(prod, [stride_idx + k])
        sig = 1.0 / (1.0 + jnp.exp(-acc))
        out_v[pl.ds(g * LANES, LANES)] = sig
        return carry

    lax.fori_loop(0, b_per_w // LANES, group, 0)
    pltpu.sync_copy(out_v, out_hbm.at[pl.ds(base, b_per_w)])


def kernel(x, user_emb_table, item_emb_table):
    batch = x.shape[0]
    embed = user_emb_table.shape[1]
    b_per_w = batch // NUM_WORKERS
    user_idx = x[:, 0]
    item_idx = x[:, 1]
    mesh = plsc.VectorSubcoreMesh(core_axis_name="c", subcore_axis_name="s")
    f = pl.kernel(
        _mf_body,
        out_type=jax.ShapeDtypeStruct((batch,), jnp.float32),
        mesh=mesh,
        compiler_params=pltpu.CompilerParams(needs_layout_passes=False,
                                             use_tc_tiling_on_sc=True),
        scratch_types=[
            pltpu.VMEM((b_per_w,), jnp.int32),
            pltpu.VMEM((b_per_w,), jnp.int32),
            pltpu.VMEM((CHUNK * 8, embed), jnp.float32),
            pltpu.VMEM((CHUNK * 8, embed), jnp.float32),
            pltpu.VMEM((b_per_w * embed,), jnp.float32),
            pltpu.VMEM((b_per_w * embed,), jnp.float32),
            pltpu.VMEM((b_per_w,), jnp.float32),
            pltpu.VMEM((LANES * LANES,), jnp.float32),
            pltpu.SemaphoreType.DMA,
            pltpu.SemaphoreType.DMA,
        ],
    )
    return f(user_idx, item_idx, user_emb_table, item_emb_table)
